# baseline (speedup 1.0000x reference)
import functools

import jax
import jax.numpy as jnp
from jax import lax
from jax.experimental import pallas as pl
from jax.experimental.pallas import tpu as pltpu
from jax.experimental.pallas import tpu_sc as plsc

_BATCH = 1024
_LAT_DIM = 64
_VOCAB = 256
_CUR_DIM = 32

_NC = 2
_NS = 16
_L = 16
_NVB = 4
_NBB = 8
_BB = _BATCH // _NBB
_VB = _VOCAB // _NVB
_NG = _BB // _L


def _slice_t_body(src_ref, dst_ref):
    dst_ref[...] = src_ref[:, _CUR_DIM % 8, :].T


_extract_t = pl.pallas_call(
    _slice_t_body,
    grid=(1,),
    in_specs=[pl.BlockSpec((_BATCH, 8, _VOCAB), lambda i: (0, _CUR_DIM // 8, 0))],
    out_specs=pl.BlockSpec((_VOCAB, _BATCH), lambda i: (0, 0)),
    out_shape=jax.ShapeDtypeStruct((_VOCAB, _BATCH), jnp.float32),
)


def _bounds_body(pt_hbm, low_hbm, upp_hbm, sj_hbm, out_low_hbm, out_upp_hbm,
                 p_v, sj_v, acc_v, res_v, low_v, upp_v, olow_v, oupp_v,
                 shared, sem):
    c = lax.axis_index("c")
    s = lax.axis_index("s")
    bbl = s // _NVB
    vb = s % _NVB
    bb = c * (_NS // _NVB) + bbl
    base = bb * _BB

    copy = pltpu.async_copy(
        pt_hbm.at[pl.ds(vb * _VB, _VB), pl.ds(base, _BB)], p_v, sem)
    pltpu.sync_copy(sj_hbm.at[pl.ds(base, _BB)], sj_v)
    copy.wait()

    sj = [sj_v[pl.ds(g * _L, _L)] for g in range(_NG)]
    alo = [jnp.zeros((_L,), jnp.float32) for _ in range(_NG)]
    aat = [jnp.zeros((_L,), jnp.float32) for _ in range(_NG)]
    zero = jnp.zeros((_L,), jnp.float32)
    for v in range(_VB):
        vg = vb * _VB + v
        for g in range(_NG):
            p = p_v[v, pl.ds(g * _L, _L)]
            alo[g] = alo[g] + jnp.where(vg < sj[g], p, zero)
            aat[g] = aat[g] + jnp.where(vg == sj[g], p, zero)
    for g in range(_NG):
        acc_v[0, pl.ds(g * _L, _L)] = alo[g]
        acc_v[1, pl.ds(g * _L, _L)] = aat[g]

    pltpu.sync_copy(acc_v, shared.at[s])
    plsc.subcore_barrier()

    @pl.when(vb == 0)
    def _():
        tlo = list(alo)
        tat = list(aat)
        pltpu.sync_copy(low_hbm.at[pl.ds(base, _BB)], low_v)
        pltpu.sync_copy(upp_hbm.at[pl.ds(base, _BB)], upp_v)
        for k in range(1, _NVB):
            pltpu.sync_copy(shared.at[s + k], res_v)
            for g in range(_NG):
                tlo[g] = tlo[g] + res_v[0, pl.ds(g * _L, _L)]
                tat[g] = tat[g] + res_v[1, pl.ds(g * _L, _L)]
        for g in range(_NG):
            cdf = tlo[g]
            pat = tat[g]
            low = low_v[pl.ds(g * _L, _L)]
            upp = upp_v[pl.ds(g * _L, _L)]
            rng = upp - low
            olow_v[pl.ds(g * _L, _L)] = low + rng * cdf
            oupp_v[pl.ds(g * _L, _L)] = low + rng * (cdf + pat)
        pltpu.sync_copy(olow_v, out_low_hbm.at[pl.ds(base, _BB)])
        pltpu.sync_copy(oupp_v, out_upp_hbm.at[pl.ds(base, _BB)])


_sc_update_bounds = functools.partial(
    pl.kernel,
    mesh=plsc.VectorSubcoreMesh(core_axis_name="c", subcore_axis_name="s"),
    compiler_params=pltpu.CompilerParams(use_tc_tiling_on_sc=True,
                                         needs_layout_passes=False),
    out_type=(jax.ShapeDtypeStruct((_BATCH,), jnp.float32),
              jax.ShapeDtypeStruct((_BATCH,), jnp.float32)),
    scratch_types=[
        pltpu.VMEM((_VB, _BB), jnp.float32),
        pltpu.VMEM((_BB,), jnp.int32),
        pltpu.VMEM((2, _BB), jnp.float32),
        pltpu.VMEM((2, _BB), jnp.float32),
        pltpu.VMEM((_BB,), jnp.float32),
        pltpu.VMEM((_BB,), jnp.float32),
        pltpu.VMEM((_BB,), jnp.float32),
        pltpu.VMEM((_BB,), jnp.float32),
        pltpu.VMEM_SHARED((_NS, 2, _BB), jnp.float32),
        pltpu.SemaphoreType.DMA,
    ],
)(_bounds_body)


def kernel(low_bound, upp_bound, softmax, s_j):
    probs_t = _extract_t(softmax)
    sj = s_j.astype(jnp.int32)
    new_low, new_upp = _sc_update_bounds(probs_t, low_bound, upp_bound, sj)
    return (new_low, new_upp)
